# baseline (speedup 1.0000x reference)
import functools

import jax
import jax.numpy as jnp
from jax import lax
from jax.experimental import pallas as pl
from jax.experimental.pallas import tpu as pltpu
from jax.experimental.pallas import tpu_sc as plsc

N = 10000
E = 320000
D = 128
K = 2
ALPHA = 0.5

NC = 2
NS = 16
NW = NC * NS
EPW = E // NW
C = 80
NCHUNK = EPW // C
NP = 10112
RPT = NP // NS

_MESH = plsc.VectorSubcoreMesh(core_axis_name="c", subcore_axis_name="s")

_NB = 3
_MAIN = NCHUNK - NCHUNK % _NB


def _edge_body(packed_hbm, hs_hbm, zeros_hbm, out_hbm,
               pk_v, sb0, sb1, sb2, db0, db1, db2, r0, r1, r2,
               s0, s1, s2, zsem, acc_sh):
    cid = lax.axis_index("c")
    sid = lax.axis_index("s")
    wid = sid * NC + cid
    sbufs = [sb0, sb1, sb2]
    dbufs = [db0, db1, db2]
    rows = [r0, r1, r2]
    sems = [s0, s1, s2]

    @pl.loop(0, RPT // 8)
    def _zero(j):
        pltpu.async_copy(zeros_hbm,
                         acc_sh.at[pl.ds(sid * RPT + j * 8, 8)], zsem)
    pltpu.sync_copy(packed_hbm.at[wid], pk_v)
    @pl.loop(0, RPT // 8)
    def _zero_wait(j):
        pltpu.make_async_copy(
            zeros_hbm, acc_sh.at[pl.ds(sid * RPT + j * 8, 8)], zsem).wait()
    plsc.subcore_barrier()

    def unpack(c, sb, db):
        for i in range(C // 16):
            x = pk_v[pl.ds(c * C + i * 16, 16)]
            sb[pl.ds(i * 16, 16)] = lax.shift_right_logical(x, 16)
            db[pl.ds(i * 16, 16)] = lax.bitwise_and(x, 0xFFFF)

    for b in range(_NB):
        unpack(b, sbufs[b], dbufs[b])
        pltpu.async_copy(hs_hbm.at[sbufs[b]], rows[b], sems[b])
    @pl.loop(0, _MAIN, step=_NB)
    def _chunk(j):
        for b in range(_NB):
            pltpu.make_async_copy(hs_hbm.at[sbufs[b]], rows[b],
                                  sems[b]).wait()
            pltpu.sync_copy(rows[b], acc_sh.at[dbufs[b]], add=True)
            nxt = j + b + _NB
            @pl.when(nxt < NCHUNK)
            def _prefetch():
                unpack(nxt, sbufs[b], dbufs[b])
                pltpu.async_copy(hs_hbm.at[sbufs[b]], rows[b], sems[b])
    for b in range(NCHUNK - _MAIN):
        pltpu.make_async_copy(hs_hbm.at[sbufs[b]], rows[b], sems[b]).wait()
        pltpu.sync_copy(rows[b], acc_sh.at[dbufs[b]], add=True)
    plsc.subcore_barrier()
    pltpu.sync_copy(acc_sh.at[pl.ds(sid * RPT, RPT)],
                    out_hbm.at[cid, pl.ds(sid * RPT, RPT)])


_edge_kernel = functools.partial(
    pl.kernel,
    out_type=jax.ShapeDtypeStruct((NC, NP, D), jnp.float32),
    mesh=_MESH,
    scratch_types=[
        pltpu.VMEM((EPW,), jnp.int32),
    ] + [pltpu.VMEM((C,), jnp.int32) for _ in range(2 * _NB)]
      + [pltpu.VMEM((C, D), jnp.float32) for _ in range(_NB)]
      + [pltpu.SemaphoreType.DMA for _ in range(_NB + 1)]
      + [pltpu.VMEM_SHARED((NP, D), jnp.float32)],
)(_edge_body)


_BN = 1000


def _mix_body(flag_ref, mp_ref, f_ref, ri_ref, normb_ref, hpre_ref,
              hs_o, ri_o, normb_o, hpre_o):
    flag = flag_ref[0, 0] > 0.5
    m = mp_ref[0] + mp_ref[1]
    f = f_ref[...]
    d = jnp.maximum(m[:, 0:1], 1.0)
    norm = lax.rsqrt(d)
    hnew = (ALPHA * (m * normb_ref[...]) + ALPHA * ri_ref[...]
            + (1.0 - ALPHA) * hpre_ref[...])
    hpre2 = jnp.where(flag, f, hnew)
    normb2 = jnp.where(flag, jnp.broadcast_to(norm, f.shape), normb_ref[...])
    ri2 = jnp.where(flag, f / d, ri_ref[...])
    hpre_o[...] = hpre2
    normb_o[...] = normb2
    ri_o[...] = ri2
    hs_o[...] = hpre2 * normb2


def _mix_kernel(flag, m_parts, features, ri, normb, h_pre):
    grid = N // _BN
    blk = pl.BlockSpec((_BN, D), lambda i: (i, 0))
    return pl.pallas_call(
        _mix_body,
        grid=(grid,),
        in_specs=[
            pl.BlockSpec((1, 1), lambda i: (0, 0)),
            pl.BlockSpec((NC, _BN, D), lambda i: (0, i, 0)),
            blk, blk, blk, blk,
        ],
        out_specs=[blk, blk, blk, blk],
        out_shape=[jax.ShapeDtypeStruct((N, D), jnp.float32)] * 4,
    )(flag, m_parts, features, ri, normb, h_pre)


def kernel(features, edge_index):
    src = edge_index[0].reshape(NW, EPW)
    dst = edge_index[1].reshape(NW, EPW)
    packed = jnp.left_shift(src, 16) | dst
    zerosD = jnp.zeros((8, D), jnp.float32)
    zerosND = jnp.zeros((N, D), jnp.float32)
    onesND = jnp.ones((N, D), jnp.float32)

    flags = jnp.concatenate(
        [jnp.ones((1, 1, 1), jnp.float32),
         jnp.zeros((K, 1, 1), jnp.float32)], axis=0)

    def step(carry, flag):
        hs, ri, normb, h_pre = carry
        m_parts = _edge_kernel(packed, hs, zerosD)
        hs2, ri2, normb2, hpre2 = _mix_kernel(
            flag, m_parts, features, ri, normb, h_pre)
        return (hs2, ri2, normb2, hpre2), None

    (hs, ri, normb, h_pre), _ = lax.scan(
        step, (onesND, zerosND, zerosND, features), flags)
    return h_pre
